# baseline (speedup 1.0000x reference)
import functools

import jax
import jax.numpy as jnp
from jax import lax
from jax.experimental import pallas as pl
from jax.experimental.pallas import tpu as pltpu
from jax.experimental.pallas import tpu_sc as plsc

E = 64
K = 8
D = 4096
T = 8192

BT = 512
NW = 32
TPW = T // NW
NG = TPW // 16
L = 16


def _gate_body(x_ref, w1_ref, w2_ref, out_ref):
    h = jnp.tanh(jnp.dot(x_ref[...], w1_ref[...]))
    out_ref[...] = jnp.dot(h, w2_ref[...])


def _gate_logits(x, W1, W2):
    return pl.pallas_call(
        _gate_body,
        grid=(T // BT,),
        in_specs=[
            pl.BlockSpec((BT, D), lambda i: (i, 0)),
            pl.BlockSpec((D, E), lambda i: (0, 0)),
            pl.BlockSpec((E, E), lambda i: (0, 0)),
        ],
        out_specs=pl.BlockSpec((BT, E), lambda i: (i, 0)),
        out_shape=jax.ShapeDtypeStruct((T, E), jnp.float32),
    )(x, W1, W2)


def _topk_body(lg_hbm, oi_hbm, os_hbm, lg_v, oi_v, os_v):
    wid = lax.axis_index("s") * 2 + lax.axis_index("c")
    base = wid * (TPW * E)
    pltpu.sync_copy(lg_hbm.at[pl.ds(base, TPW * E)], lg_v)

    lane = lax.iota(jnp.int32, L)

    def group_body(g, _):
        rowsE = (g * L + lane) * E
        rows8 = (g * L + lane) * K

        neg_inf = jnp.full((L,), -jnp.inf, jnp.float32)
        zero_i = jnp.zeros((L,), jnp.int32)
        t_init = tuple(neg_inf for _ in range(K))
        i_init = tuple(zero_i for _ in range(K))

        def expert_body(e, carry):
            ts, is_ = carry
            v = plsc.load_gather(lg_v, [rowsE + e])
            iv = jnp.full((L,), 0, jnp.int32) + e
            ins = jnp.zeros((L,), jnp.bool_)
            new_ts, new_is = [], []
            for r in range(K):
                gt = v > ts[r]
                cond = jnp.logical_or(ins, gt)
                new_ts.append(jnp.where(cond, v, ts[r]))
                v = jnp.where(cond, ts[r], v)
                new_is.append(jnp.where(cond, iv, is_[r]))
                iv = jnp.where(cond, is_[r], iv)
                ins = cond
            return tuple(new_ts), tuple(new_is)

        ts, is_ = lax.fori_loop(0, E, expert_body, (t_init, i_init))

        exps = [jnp.exp(t - ts[0]) for t in ts]
        s = exps[0]
        for r in range(1, K):
            s = s + exps[r]
        inv = 1.0 / s
        for r in range(K):
            plsc.store_scatter(oi_v, [rows8 + r], is_[r])
            plsc.store_scatter(os_v, [rows8 + r], exps[r] * inv)
        return _

    lax.fori_loop(0, NG, group_body, None)

    obase = wid * (TPW * K)
    pltpu.sync_copy(oi_v, oi_hbm.at[pl.ds(obase, TPW * K)])
    pltpu.sync_copy(os_v, os_hbm.at[pl.ds(obase, TPW * K)])


@functools.cache
def _topk_sc():
    return pl.kernel(
        _topk_body,
        out_type=(
            jax.ShapeDtypeStruct((T * K,), jnp.int32),
            jax.ShapeDtypeStruct((T * K,), jnp.float32),
        ),
        mesh=plsc.VectorSubcoreMesh(core_axis_name="c", subcore_axis_name="s"),
        compiler_params=pltpu.CompilerParams(needs_layout_passes=False),
        scratch_types=[
            pltpu.VMEM((TPW * E,), jnp.float32),
            pltpu.VMEM((TPW * K,), jnp.int32),
            pltpu.VMEM((TPW * K,), jnp.float32),
        ],
    )


def kernel(x, W1, W2):
    logits = _gate_logits(x, W1, W2)
    idx_flat, scr_flat = _topk_sc()(logits.reshape(T * E))
    return idx_flat.reshape(T, K), scr_flat.reshape(T, K)
